# initial kernel scaffold (unmeasured)
import jax
import jax.numpy as jnp
from jax import lax
from jax.experimental import pallas as pl
from jax.experimental.pallas import tpu as pltpu

N_DEV = 4
N_TOK = 2048
D = 1024
E_LOCAL = 8
CAP = 51
SLOTS_PER_E = 64
N_SLOT = E_LOCAL * SLOTS_PER_E
ROWS_PER_DEV = N_TOK // N_DEV


def kernel(x, router_W, route_idx, expert_W):
    del router_W

    my = lax.axis_index("i")

    e = route_idx[:, 0]
    onehot = (e[:, None] == jnp.arange(32, dtype=e.dtype)[None, :]).astype(jnp.int32)
    pos = jnp.take_along_axis(jnp.cumsum(onehot, axis=0), e[:, None], axis=1)[:, 0] - 1
    le = e - E_LOCAL * my
    valid = (le >= 0) & (le < E_LOCAL) & (pos < CAP)
    slot = jnp.where(valid, le * SLOTS_PER_E + pos, N_SLOT)
    tok = (
        jnp.full((N_SLOT + 1,), N_TOK, jnp.int32)
        .at[slot]
        .set(jnp.arange(N_TOK, dtype=jnp.int32))
    )[:N_SLOT]
    tok2d = tok.reshape(N_SLOT, 1)

    def body(x_ref, tok_ref, w_ref, out_ref, comp_ref, contrib_ref, recv_ref,
             send_sems, recv_sems):
        my_pos = lax.axis_index("i")

        barrier_sem = pltpu.get_barrier_semaphore()
        for k in range(1, N_DEV):
            pl.semaphore_signal(
                barrier_sem, inc=1,
                device_id=((my_pos + k) % N_DEV,),
                device_id_type=pl.DeviceIdType.MESH,
            )
        pl.semaphore_wait(barrier_sem, N_DEV - 1)

        iota_t = lax.broadcasted_iota(jnp.int32, (N_SLOT, N_TOK), 1)
        s_mat = (tok_ref[:, :] == iota_t).astype(jnp.bfloat16)

        xg = jnp.dot(
            s_mat, x_ref[:, :].astype(jnp.bfloat16),
            preferred_element_type=jnp.float32,
        ).astype(jnp.bfloat16)

        for lei in range(E_LOCAL):
            a = xg[lei * SLOTS_PER_E:(lei + 1) * SLOTS_PER_E, :]
            w = w_ref[lei, :, :].astype(jnp.bfloat16)
            comp_ref[pl.ds(lei * SLOTS_PER_E, SLOTS_PER_E), :] = jnp.dot(
                a, w, preferred_element_type=jnp.float32
            ).astype(jnp.bfloat16)

        comp = comp_ref[:, :]

        rdmas = []
        for k in range(N_DEV):
            d = (my_pos + k) % N_DEV
            s_d = lax.dynamic_slice_in_dim(s_mat, d * ROWS_PER_DEV, ROWS_PER_DEV, axis=1)
            blk = lax.dot_general(
                s_d, comp,
                dimension_numbers=(((0,), (0,)), ((), ())),
                preferred_element_type=jnp.float32,
            ).astype(jnp.bfloat16)
            contrib_ref[k] = blk
            if k > 0:
                rdma = pltpu.make_async_remote_copy(
                    src_ref=contrib_ref.at[k],
                    dst_ref=recv_ref.at[3 - k],
                    send_sem=send_sems.at[k - 1],
                    recv_sem=recv_sems.at[3 - k],
                    device_id=(d,),
                    device_id_type=pl.DeviceIdType.MESH,
                )
                rdma.start()
                rdmas.append(rdma)

        acc = contrib_ref[0].astype(jnp.float32)
        for j in range(N_DEV - 1):
            recv = pltpu.make_async_remote_copy(
                src_ref=contrib_ref.at[0],
                dst_ref=recv_ref.at[j],
                send_sem=send_sems.at[0],
                recv_sem=recv_sems.at[j],
                device_id=(my_pos,),
                device_id_type=pl.DeviceIdType.MESH,
            )
            recv.wait_recv()
            acc = acc + recv_ref[j].astype(jnp.float32)
        out_ref[:, :] = acc

        for rdma in rdmas:
            rdma.wait_send()

    out = pl.pallas_call(
        body,
        out_shape=jax.ShapeDtypeStruct((ROWS_PER_DEV, D), jnp.float32),
        in_specs=[
            pl.BlockSpec(memory_space=pltpu.VMEM),
            pl.BlockSpec(memory_space=pltpu.VMEM),
            pl.BlockSpec(memory_space=pltpu.VMEM),
        ],
        out_specs=pl.BlockSpec(memory_space=pltpu.VMEM),
        scratch_shapes=[
            pltpu.VMEM((N_SLOT, D), jnp.bfloat16),
            pltpu.VMEM((N_DEV, ROWS_PER_DEV, D), jnp.bfloat16),
            pltpu.VMEM((N_DEV - 1, ROWS_PER_DEV, D), jnp.bfloat16),
            pltpu.SemaphoreType.DMA((N_DEV - 1,)),
            pltpu.SemaphoreType.DMA((N_DEV - 1,)),
        ],
        compiler_params=pltpu.CompilerParams(collective_id=0),
    )(x, tok2d, expert_W)
    return out


# baseline (device time: 80164 ns/iter reference)
import jax
import jax.numpy as jnp
from jax import lax
from jax.experimental import pallas as pl
from jax.experimental.pallas import tpu as pltpu

N_DEV = 4
N_TOK = 2048
D = 1024
E_LOCAL = 8
CAP = 51
SLOTS_PER_E = 64
N_SLOT = E_LOCAL * SLOTS_PER_E
ROWS_PER_DEV = N_TOK // N_DEV


def kernel(x, router_W, route_idx, expert_W):
    del router_W

    my = lax.axis_index("i")

    e = route_idx[:, 0]
    onehot = (e[:, None] == jnp.arange(32, dtype=e.dtype)[None, :]).astype(jnp.int32)
    pos = jnp.take_along_axis(jnp.cumsum(onehot, axis=0), e[:, None], axis=1)[:, 0] - 1
    le = e - E_LOCAL * my
    valid = (le >= 0) & (le < E_LOCAL) & (pos < CAP)
    slot = jnp.where(valid, le * SLOTS_PER_E + pos, N_SLOT)
    tok = (
        jnp.full((N_SLOT + 1,), N_TOK, jnp.int32)
        .at[slot]
        .set(jnp.arange(N_TOK, dtype=jnp.int32))
    )[:N_SLOT]
    tok2d = tok.reshape(N_SLOT, 1)

    def body(x_ref, tok_ref, w_ref, out_ref, comp_ref, contrib_ref, recv_ref,
             send_sems, recv_sems):
        my_pos = lax.axis_index("i")

        barrier_sem = pltpu.get_barrier_semaphore()
        for k in range(1, N_DEV):
            pl.semaphore_signal(
                barrier_sem, inc=1,
                device_id=((my_pos + k) % N_DEV,),
                device_id_type=pl.DeviceIdType.MESH,
            )
        pl.semaphore_wait(barrier_sem, N_DEV - 1)

        iota_t = lax.broadcasted_iota(jnp.int32, (N_SLOT, N_TOK), 1)
        s_mat = (tok_ref[:, :] == iota_t).astype(jnp.bfloat16)

        xg = jnp.dot(
            s_mat, x_ref[:, :].astype(jnp.bfloat16),
            preferred_element_type=jnp.float32,
        ).astype(jnp.bfloat16)

        for lei in range(E_LOCAL):
            a = xg[lei * SLOTS_PER_E:(lei + 1) * SLOTS_PER_E, :]
            w = w_ref[lei, :, :].astype(jnp.bfloat16)
            comp_ref[pl.ds(lei * SLOTS_PER_E, SLOTS_PER_E), :] = jnp.dot(
                a, w, preferred_element_type=jnp.float32
            ).astype(jnp.bfloat16)

        comp = comp_ref[:, :]

        iota_blk = lax.broadcasted_iota(jnp.int32, (N_SLOT, ROWS_PER_DEV), 1)
        rdmas = []
        for k in range(N_DEV):
            d = (my_pos + k) % N_DEV
            s_d = (tok_ref[:, :] == iota_blk + d * ROWS_PER_DEV).astype(jnp.bfloat16)
            blk = lax.dot_general(
                s_d, comp,
                dimension_numbers=(((0,), (0,)), ((), ())),
                preferred_element_type=jnp.float32,
            ).astype(jnp.bfloat16)
            contrib_ref[k] = blk
            if k > 0:
                rdma = pltpu.make_async_remote_copy(
                    src_ref=contrib_ref.at[k],
                    dst_ref=recv_ref.at[3 - k],
                    send_sem=send_sems.at[k - 1],
                    recv_sem=recv_sems.at[3 - k],
                    device_id=(d,),
                    device_id_type=pl.DeviceIdType.MESH,
                )
                rdma.start()
                rdmas.append(rdma)

        acc = contrib_ref[0].astype(jnp.float32)
        for j in range(N_DEV - 1):
            recv = pltpu.make_async_remote_copy(
                src_ref=contrib_ref.at[0],
                dst_ref=recv_ref.at[j],
                send_sem=send_sems.at[0],
                recv_sem=recv_sems.at[j],
                device_id=(my_pos,),
                device_id_type=pl.DeviceIdType.MESH,
            )
            recv.wait_recv()
            acc = acc + recv_ref[j].astype(jnp.float32)
        out_ref[:, :] = acc

        for rdma in rdmas:
            rdma.wait_send()

    out = pl.pallas_call(
        body,
        out_shape=jax.ShapeDtypeStruct((ROWS_PER_DEV, D), jnp.float32),
        in_specs=[
            pl.BlockSpec(memory_space=pltpu.VMEM),
            pl.BlockSpec(memory_space=pltpu.VMEM),
            pl.BlockSpec(memory_space=pltpu.VMEM),
        ],
        out_specs=pl.BlockSpec(memory_space=pltpu.VMEM),
        scratch_shapes=[
            pltpu.VMEM((N_SLOT, D), jnp.bfloat16),
            pltpu.VMEM((N_DEV, ROWS_PER_DEV, D), jnp.bfloat16),
            pltpu.VMEM((N_DEV - 1, ROWS_PER_DEV, D), jnp.bfloat16),
            pltpu.SemaphoreType.DMA((N_DEV - 1,)),
            pltpu.SemaphoreType.DMA((N_DEV - 1,)),
        ],
        compiler_params=pltpu.CompilerParams(
            collective_id=0,
            vmem_limit_bytes=100 * 1024 * 1024,
        ),
    )(x, tok2d, expert_W)
    return out


# device time: 64234 ns/iter; 1.2480x vs baseline; 1.2480x over previous
import jax
import jax.numpy as jnp
from jax import lax
from jax.experimental import pallas as pl
from jax.experimental.pallas import tpu as pltpu

N_DEV = 4
N_TOK = 2048
D = 1024
N_EXP = 32
E_LOCAL = 8
CAP = 51
SLOTS_PER_E = 64
N_SLOT = E_LOCAL * SLOTS_PER_E
ROWS_PER_DEV = N_TOK // N_DEV


def kernel(x, router_W, route_idx, expert_W):
    del router_W

    def body(x_ref, idx_ref, w_ref, out_ref, comp_ref, contrib_ref, recv_ref,
             send_sems, recv_sems):
        my_pos = lax.axis_index("i")

        barrier_sem = pltpu.get_barrier_semaphore()
        for k in range(1, N_DEV):
            pl.semaphore_signal(
                barrier_sem, inc=1,
                device_id=((my_pos + k) % N_DEV,),
                device_id_type=pl.DeviceIdType.MESH,
            )
        pl.semaphore_wait(barrier_sem, N_DEV - 1)

        e = idx_ref[:, :]
        onehot = (e == lax.broadcasted_iota(jnp.int32, (N_TOK, N_EXP), 1)
                  ).astype(jnp.bfloat16)
        tri = (lax.broadcasted_iota(jnp.int32, (N_TOK, N_TOK), 1)
               <= lax.broadcasted_iota(jnp.int32, (N_TOK, N_TOK), 0)
               ).astype(jnp.bfloat16)
        counts = jnp.dot(tri, onehot, preferred_element_type=jnp.float32)
        pos = jnp.sum(counts * onehot.astype(jnp.float32), axis=1,
                      keepdims=True).astype(jnp.int32) - 1

        le = e - E_LOCAL * my_pos
        valid = (le >= 0) & (le < E_LOCAL) & (pos < CAP)
        slot = jnp.where(valid, le * SLOTS_PER_E + pos, N_SLOT)

        s_t = (slot == lax.broadcasted_iota(jnp.int32, (N_TOK, N_SLOT), 1)
               ).astype(jnp.bfloat16)

        xg = lax.dot_general(
            s_t, x_ref[:, :].astype(jnp.bfloat16),
            dimension_numbers=(((0,), (0,)), ((), ())),
            preferred_element_type=jnp.float32,
        ).astype(jnp.bfloat16)

        for lei in range(E_LOCAL):
            a = xg[lei * SLOTS_PER_E:(lei + 1) * SLOTS_PER_E, :]
            w = w_ref[lei, :, :].astype(jnp.bfloat16)
            comp_ref[pl.ds(lei * SLOTS_PER_E, SLOTS_PER_E), :] = jnp.dot(
                a, w, preferred_element_type=jnp.float32
            ).astype(jnp.bfloat16)

        comp = comp_ref[:, :]

        for d in range(N_DEV):
            s_blk = s_t[d * ROWS_PER_DEV:(d + 1) * ROWS_PER_DEV, :]
            contrib_ref[d] = jnp.dot(
                s_blk, comp, preferred_element_type=jnp.float32
            ).astype(jnp.bfloat16)

        rdmas = []
        for k in range(1, N_DEV):
            d = (my_pos + k) % N_DEV
            rdma = pltpu.make_async_remote_copy(
                src_ref=contrib_ref.at[d],
                dst_ref=recv_ref.at[3 - k],
                send_sem=send_sems.at[k - 1],
                recv_sem=recv_sems.at[3 - k],
                device_id=(d,),
                device_id_type=pl.DeviceIdType.MESH,
            )
            rdma.start()
            rdmas.append(rdma)

        acc = contrib_ref[my_pos].astype(jnp.float32)
        for j in range(N_DEV - 1):
            recv = pltpu.make_async_remote_copy(
                src_ref=contrib_ref.at[0],
                dst_ref=recv_ref.at[j],
                send_sem=send_sems.at[0],
                recv_sem=recv_sems.at[j],
                device_id=(my_pos,),
                device_id_type=pl.DeviceIdType.MESH,
            )
            recv.wait_recv()
            acc = acc + recv_ref[j].astype(jnp.float32)
        out_ref[:, :] = acc

        for rdma in rdmas:
            rdma.wait_send()

    out = pl.pallas_call(
        body,
        out_shape=jax.ShapeDtypeStruct((ROWS_PER_DEV, D), jnp.float32),
        in_specs=[
            pl.BlockSpec(memory_space=pltpu.VMEM),
            pl.BlockSpec(memory_space=pltpu.VMEM),
            pl.BlockSpec(memory_space=pltpu.VMEM),
        ],
        out_specs=pl.BlockSpec(memory_space=pltpu.VMEM),
        scratch_shapes=[
            pltpu.VMEM((N_SLOT, D), jnp.bfloat16),
            pltpu.VMEM((N_DEV, ROWS_PER_DEV, D), jnp.bfloat16),
            pltpu.VMEM((N_DEV - 1, ROWS_PER_DEV, D), jnp.bfloat16),
            pltpu.SemaphoreType.DMA((N_DEV - 1,)),
            pltpu.SemaphoreType.DMA((N_DEV - 1,)),
        ],
        compiler_params=pltpu.CompilerParams(
            collective_id=0,
            vmem_limit_bytes=110 * 1024 * 1024,
        ),
    )(x, route_idx, expert_W)
    return out


# device time: 61664 ns/iter; 1.3000x vs baseline; 1.0417x over previous
import jax
import jax.numpy as jnp
from jax import lax
from jax.experimental import pallas as pl
from jax.experimental.pallas import tpu as pltpu

N_DEV = 4
N_TOK = 2048
D = 1024
N_EXP = 32
E_LOCAL = 8
CAP = 51
SLOTS_PER_E = 64
N_SLOT = E_LOCAL * SLOTS_PER_E
ROWS_PER_DEV = N_TOK // N_DEV
N_PHASE = 2
D_PH = D // N_PHASE


def kernel(x, router_W, route_idx, expert_W):
    del router_W

    def body(x_ref, idx_ref, w_ref, out_ref, comp_ref, contrib_ref, recv_ref,
             send_sems, recv_sems):
        my_pos = lax.axis_index("i")

        barrier_sem = pltpu.get_barrier_semaphore()
        for k in range(1, N_DEV):
            pl.semaphore_signal(
                barrier_sem, inc=1,
                device_id=((my_pos + k) % N_DEV,),
                device_id_type=pl.DeviceIdType.MESH,
            )
        pl.semaphore_wait(barrier_sem, N_DEV - 1)

        e = idx_ref[:, :]
        onehot = (e == lax.broadcasted_iota(jnp.int32, (N_TOK, N_EXP), 1)
                  ).astype(jnp.bfloat16)
        tri = (lax.broadcasted_iota(jnp.int32, (N_TOK, N_TOK), 1)
               <= lax.broadcasted_iota(jnp.int32, (N_TOK, N_TOK), 0)
               ).astype(jnp.bfloat16)
        counts = jnp.dot(tri, onehot, preferred_element_type=jnp.float32)
        pos = jnp.sum(counts * onehot.astype(jnp.float32), axis=1,
                      keepdims=True).astype(jnp.int32) - 1

        le = e - E_LOCAL * my_pos
        valid = (le >= 0) & (le < E_LOCAL) & (pos < CAP)
        slot = jnp.where(valid, le * SLOTS_PER_E + pos, N_SLOT)

        s_t = (slot == lax.broadcasted_iota(jnp.int32, (N_TOK, N_SLOT), 1)
               ).astype(jnp.bfloat16)

        xg = lax.dot_general(
            s_t, x_ref[:, :].astype(jnp.bfloat16),
            dimension_numbers=(((0,), (0,)), ((), ())),
            preferred_element_type=jnp.float32,
        ).astype(jnp.bfloat16)

        rdmas = []
        for h in range(N_PHASE):
            for lei in range(E_LOCAL):
                a = xg[lei * SLOTS_PER_E:(lei + 1) * SLOTS_PER_E, :]
                w = w_ref[lei, :, pl.ds(h * D_PH, D_PH)].astype(jnp.bfloat16)
                comp_ref[pl.ds(lei * SLOTS_PER_E, SLOTS_PER_E),
                         pl.ds(h * D_PH, D_PH)] = jnp.dot(
                    a, w, preferred_element_type=jnp.float32
                ).astype(jnp.bfloat16)

            comp_h = comp_ref[:, pl.ds(h * D_PH, D_PH)]

            for d in range(N_DEV):
                s_blk = s_t[d * ROWS_PER_DEV:(d + 1) * ROWS_PER_DEV, :]
                contrib_ref[h, d] = jnp.dot(
                    s_blk, comp_h, preferred_element_type=jnp.float32
                ).astype(jnp.bfloat16)

            for k in range(1, N_DEV):
                d = (my_pos + k) % N_DEV
                rdma = pltpu.make_async_remote_copy(
                    src_ref=contrib_ref.at[h, d],
                    dst_ref=recv_ref.at[h, 3 - k],
                    send_sem=send_sems.at[h, k - 1],
                    recv_sem=recv_sems.at[h, 3 - k],
                    device_id=(d,),
                    device_id_type=pl.DeviceIdType.MESH,
                )
                rdma.start()
                rdmas.append(rdma)

        for h in range(N_PHASE):
            acc = contrib_ref[h, my_pos].astype(jnp.float32)
            for j in range(N_DEV - 1):
                recv = pltpu.make_async_remote_copy(
                    src_ref=contrib_ref.at[h, 0],
                    dst_ref=recv_ref.at[h, j],
                    send_sem=send_sems.at[h, 0],
                    recv_sem=recv_sems.at[h, j],
                    device_id=(my_pos,),
                    device_id_type=pl.DeviceIdType.MESH,
                )
                recv.wait_recv()
                acc = acc + recv_ref[h, j].astype(jnp.float32)
            out_ref[:, pl.ds(h * D_PH, D_PH)] = acc

        for rdma in rdmas:
            rdma.wait_send()

    out = pl.pallas_call(
        body,
        out_shape=jax.ShapeDtypeStruct((ROWS_PER_DEV, D), jnp.float32),
        in_specs=[
            pl.BlockSpec(memory_space=pltpu.VMEM),
            pl.BlockSpec(memory_space=pltpu.VMEM),
            pl.BlockSpec(memory_space=pltpu.VMEM),
        ],
        out_specs=pl.BlockSpec(memory_space=pltpu.VMEM),
        scratch_shapes=[
            pltpu.VMEM((N_SLOT, D), jnp.bfloat16),
            pltpu.VMEM((N_PHASE, N_DEV, ROWS_PER_DEV, D_PH),
                       jnp.bfloat16),
            pltpu.VMEM((N_PHASE, N_DEV - 1, ROWS_PER_DEV, D_PH),
                       jnp.bfloat16),
            pltpu.SemaphoreType.DMA((N_PHASE, N_DEV - 1)),
            pltpu.SemaphoreType.DMA((N_PHASE, N_DEV - 1)),
        ],
        compiler_params=pltpu.CompilerParams(
            collective_id=0,
            vmem_limit_bytes=110 * 1024 * 1024,
        ),
    )(x, route_idx, expert_W)
    return out


# device time: 44564 ns/iter; 1.7989x vs baseline; 1.3837x over previous
import jax
import jax.numpy as jnp
from jax import lax
from jax.experimental import pallas as pl
from jax.experimental.pallas import tpu as pltpu

N_DEV = 4
N_TOK = 2048
D = 1024
N_EXP = 32
E_LOCAL = 8
CAP = 51
SLOTS_PER_E = 64
N_SLOT = E_LOCAL * SLOTS_PER_E
ROWS_PER_DEV = N_TOK // N_DEV
N_HALF = 2
D_HF = D // N_HALF
N_PHASE = 4
D_PH = D // N_PHASE


def kernel(x, router_W, route_idx, expert_W):
    del router_W

    def body(x_hbm, idx_ref, w_hbm, out_ref, x_vmem, wbuf, comp_ref,
             contrib_ref, recv_ref, x_sem, w_sems, send_sems, recv_sems):
        my_pos = lax.axis_index("i")

        x_copy = pltpu.make_async_copy(x_hbm, x_vmem, x_sem)
        x_copy.start()
        w_copies = {}

        def start_w_quarter(h):
            for lei in range(E_LOCAL):
                c = pltpu.make_async_copy(
                    w_hbm.at[lei, :, pl.ds(h * D_PH, D_PH)],
                    wbuf.at[lei, :, pl.ds(h * D_PH, D_PH)],
                    w_sems.at[lei, h],
                )
                c.start()
                w_copies[lei, h] = c

        start_w_quarter(0)

        barrier_sem = pltpu.get_barrier_semaphore()
        for k in range(1, N_DEV):
            pl.semaphore_signal(
                barrier_sem, inc=1,
                device_id=((my_pos + k) % N_DEV,),
                device_id_type=pl.DeviceIdType.MESH,
            )
        pl.semaphore_wait(barrier_sem, N_DEV - 1)

        e = idx_ref[:, :]
        onehot = (e == lax.broadcasted_iota(jnp.int32, (N_TOK, N_EXP), 1)
                  ).astype(jnp.int32)
        counts = onehot
        sh = 1
        while sh < N_TOK:
            counts = counts + jnp.concatenate(
                [jnp.zeros((sh, N_EXP), jnp.int32), counts[:N_TOK - sh, :]],
                axis=0)
            sh *= 2
        pos = jnp.sum(counts * onehot, axis=1, keepdims=True) - 1

        le = e - E_LOCAL * my_pos
        valid = (le >= 0) & (le < E_LOCAL) & (pos < CAP)
        slot = jnp.where(valid, le * SLOTS_PER_E + pos, N_SLOT)

        s_t = (slot == lax.broadcasted_iota(jnp.int32, (N_TOK, N_SLOT), 1)
               ).astype(jnp.bfloat16)

        x_copy.wait()
        xg = lax.dot_general(
            s_t, x_vmem[:, :].astype(jnp.bfloat16),
            dimension_numbers=(((0,), (0,)), ((), ())),
            preferred_element_type=jnp.float32,
        ).astype(jnp.bfloat16)

        for h in range(N_PHASE):
            for lei in range(E_LOCAL):
                w_copies[lei, h].wait()
                a = xg[lei * SLOTS_PER_E:(lei + 1) * SLOTS_PER_E, :]
                w = wbuf[lei, :, pl.ds(h * D_PH, D_PH)].astype(jnp.bfloat16)
                comp_ref[pl.ds(lei * SLOTS_PER_E, SLOTS_PER_E),
                         pl.ds(h * D_PH, D_PH)] = jnp.dot(
                    a, w, preferred_element_type=jnp.float32
                ).astype(jnp.bfloat16)

            comp_h = comp_ref[:, pl.ds(h * D_PH, D_PH)]
            for d in range(N_DEV):
                s_blk = s_t[d * ROWS_PER_DEV:(d + 1) * ROWS_PER_DEV, :]
                contrib_ref[h, d] = jnp.dot(
                    s_blk, comp_h, preferred_element_type=jnp.float32
                ).astype(jnp.bfloat16)
            for k in range(1, N_DEV):
                d = (my_pos + k) % N_DEV
                rdma = pltpu.make_async_remote_copy(
                    src_ref=contrib_ref.at[h, d],
                    dst_ref=recv_ref.at[h, 3 - k],
                    send_sem=send_sems.at[h, k - 1],
                    recv_sem=recv_sems.at[h, 3 - k],
                    device_id=(d,),
                    device_id_type=pl.DeviceIdType.MESH,
                )
                rdma.start()
            if h + 1 < N_PHASE:
                start_w_quarter(h + 1)

        for h in range(N_PHASE):
            acc = contrib_ref[h, my_pos].astype(jnp.float32)
            for j in range(N_DEV - 1):
                recv = pltpu.make_async_remote_copy(
                    src_ref=contrib_ref.at[h, 0],
                    dst_ref=recv_ref.at[h, j],
                    send_sem=send_sems.at[h, 0],
                    recv_sem=recv_sems.at[h, j],
                    device_id=(my_pos,),
                    device_id_type=pl.DeviceIdType.MESH,
                )
                recv.wait_recv()
                acc = acc + recv_ref[h, j].astype(jnp.float32)
            out_ref[:, pl.ds(h * D_PH, D_PH)] = acc.astype(jnp.bfloat16)

        for h in range(N_PHASE):
            for k in range(1, N_DEV):
                d = (my_pos + k) % N_DEV
                rdma = pltpu.make_async_remote_copy(
                    src_ref=contrib_ref.at[h, d],
                    dst_ref=recv_ref.at[h, 3 - k],
                    send_sem=send_sems.at[h, k - 1],
                    recv_sem=recv_sems.at[h, 3 - k],
                    device_id=(d,),
                    device_id_type=pl.DeviceIdType.MESH,
                )
                rdma.wait_send()

    out = pl.pallas_call(
        body,
        out_shape=jax.ShapeDtypeStruct((ROWS_PER_DEV, D), jnp.bfloat16),
        in_specs=[
            pl.BlockSpec(memory_space=pl.ANY),
            pl.BlockSpec(memory_space=pltpu.VMEM),
            pl.BlockSpec(memory_space=pl.ANY),
        ],
        out_specs=pl.BlockSpec(memory_space=pltpu.VMEM),
        scratch_shapes=[
            pltpu.VMEM((N_TOK, D), jnp.float32),
            pltpu.VMEM((E_LOCAL, D, D), jnp.float32),
            pltpu.VMEM((N_SLOT, D), jnp.bfloat16),
            pltpu.VMEM((N_PHASE, N_DEV, ROWS_PER_DEV, D_PH),
                       jnp.bfloat16),
            pltpu.VMEM((N_PHASE, N_DEV - 1, ROWS_PER_DEV, D_PH),
                       jnp.bfloat16),
            pltpu.SemaphoreType.DMA,
            pltpu.SemaphoreType.DMA((E_LOCAL, N_PHASE)),
            pltpu.SemaphoreType.DMA((N_PHASE, N_DEV - 1)),
            pltpu.SemaphoreType.DMA((N_PHASE, N_DEV - 1)),
        ],
        compiler_params=pltpu.CompilerParams(
            collective_id=0,
            vmem_limit_bytes=110 * 1024 * 1024,
        ),
    )(x, route_idx, expert_W)
    return out
